# baseline (speedup 1.0000x reference)
import functools

import jax
import jax.numpy as jnp
from jax import lax
from jax.experimental import pallas as pl
from jax.experimental.pallas import tpu as pltpu
from jax.experimental.pallas import tpu_sc as plsc


def _body(x_hbm, w_hbm, b_hbm, out_hbm, x_s, w_s, b_s, out_s, sem):
    cp_x = pltpu.async_copy(x_hbm, x_s, sem)
    cp_w = pltpu.async_copy(w_hbm, w_s, sem)
    cp_b = pltpu.async_copy(b_hbm, b_s, sem)
    cp_x.wait()
    cp_w.wait()
    cp_b.wait()

    flat = (
        x_s[2, 0] + 2.0 * x_s[3, 0],
        x_s[2, 1] + 2.0 * x_s[3, 1],
        x_s[0, 0],
        x_s[0, 1],
        3.0 * x_s[3, 0],
        3.0 * x_s[3, 1],
    )
    for j in range(4):
        acc = b_s[j]
        for k in range(6):
            acc = acc + w_s[j, k] * flat[k]
        out_s[j] = acc
    pltpu.sync_copy(out_s, out_hbm)


@functools.partial(
    pl.kernel,
    out_type=jax.ShapeDtypeStruct((4,), jnp.float32),
    mesh=plsc.ScalarSubcoreMesh(axis_name="c", num_cores=1),
    scratch_types=[
        pltpu.SMEM((4, 2), jnp.float32),
        pltpu.SMEM((4, 6), jnp.float32),
        pltpu.SMEM((4,), jnp.float32),
        pltpu.SMEM((4,), jnp.float32),
        pltpu.SemaphoreType.DMA,
    ],
)
def _sc_kernel(x_hbm, w_hbm, b_hbm, out_hbm, x_s, w_s, b_s, out_s, sem):
    _body(x_hbm, w_hbm, b_hbm, out_hbm, x_s, w_s, b_s, out_s, sem)


def kernel(x, W, b):
    return _sc_kernel(x, W, b)


# speedup vs baseline: 1.0009x; 1.0009x over previous
import functools

import jax
import jax.numpy as jnp
from jax.experimental import pallas as pl
from jax.experimental.pallas import tpu as pltpu
from jax.experimental.pallas import tpu_sc as plsc


def _body(x_hbm, w_hbm, b_hbm, out_hbm, x_s, w_s, b_s, out_s, sem):
    cp_x = pltpu.async_copy(x_hbm, x_s, sem)
    cp_w = pltpu.async_copy(w_hbm, w_s, sem)
    cp_b = pltpu.async_copy(b_hbm, b_s, sem)
    cp_x.wait()
    cp_w.wait()
    cp_b.wait()

    flat = (
        x_s[2, 0] + 2.0 * x_s[3, 0],
        x_s[2, 1] + 2.0 * x_s[3, 1],
        x_s[0, 0],
        x_s[0, 1],
        3.0 * x_s[3, 0],
        3.0 * x_s[3, 1],
    )
    for j in range(4):
        acc = b_s[j]
        for k in range(6):
            acc = acc + w_s[j, k] * flat[k]
        out_s[j] = acc
    pltpu.sync_copy(out_s, out_hbm)


@functools.partial(
    pl.kernel,
    out_type=jax.ShapeDtypeStruct((4,), jnp.float32),
    mesh=plsc.ScalarSubcoreMesh(axis_name="c", num_cores=1),
    scratch_types=[
        pltpu.SMEM((4, 2), jnp.float32),
        pltpu.SMEM((4, 6), jnp.float32),
        pltpu.SMEM((4,), jnp.float32),
        pltpu.SMEM((4,), jnp.float32),
        pltpu.SemaphoreType.DMA,
    ],
)
def _sc_kernel(x_hbm, w_hbm, b_hbm, out_hbm, x_s, w_s, b_s, out_s, sem):
    _body(x_hbm, w_hbm, b_hbm, out_hbm, x_s, w_s, b_s, out_s, sem)


def kernel(x, W, b):
    return _sc_kernel(x, W, b)
